# initial kernel scaffold (unmeasured)
import jax
import jax.numpy as jnp
from jax import lax
from jax.experimental import pallas as pl
from jax.experimental.pallas import tpu as pltpu

N_DEV = 8


def kernel(A, B):
    A = A.astype(jnp.bfloat16)
    B = B.astype(jnp.bfloat16)
    m, _ = A.shape
    _, n = B.shape
    chunk = m // N_DEV
    rs_rows = m // 2 + m // 4 + m // 8

    def body(a_ref, b_ref, out_ref, recv_ref, send_sems, recv_sems):
        p = lax.axis_index("i")
        v = p ^ ((p >> 1) & 1)

        def to_logical(vq):
            return vq ^ ((vq >> 1) & 1)

        partners = [to_logical(v ^ (1 << b)) for b in range(3)]

        barrier = pltpu.get_barrier_semaphore()
        for b in range(3):
            pl.semaphore_signal(
                barrier, inc=1,
                device_id=(partners[b],),
                device_id_type=pl.DeviceIdType.MESH,
            )
        pl.semaphore_wait(barrier, 3)

        out_ref[...] = jnp.dot(
            a_ref[...], b_ref[...], preferred_element_type=jnp.float32
        ).astype(jnp.bfloat16)

        cur = jnp.int32(0)
        recv_off = 0
        for idx, bit in enumerate((2, 1, 0)):
            half = m >> (idx + 1)
            mybit = (v >> bit) & 1
            keep = cur + mybit * half
            give = cur + (1 - mybit) * half
            rdma = pltpu.make_async_remote_copy(
                src_ref=out_ref.at[pl.ds(give, half), :],
                dst_ref=recv_ref.at[pl.ds(recv_off, half), :],
                send_sem=send_sems.at[idx],
                recv_sem=recv_sems.at[idx],
                device_id=(partners[bit],),
                device_id_type=pl.DeviceIdType.MESH,
            )
            rdma.start()
            rdma.wait()
            out_ref[pl.ds(keep, half), :] = (
                out_ref[pl.ds(keep, half), :]
                + recv_ref[pl.ds(recv_off, half), :]
            )
            cur = keep
            recv_off += half


        for idx in range(3):
            size = chunk << idx
            start = ((v >> idx) << idx) * chunk
            rdma = pltpu.make_async_remote_copy(
                src_ref=out_ref.at[pl.ds(start, size), :],
                dst_ref=out_ref.at[pl.ds(start, size), :],
                send_sem=send_sems.at[3 + idx],
                recv_sem=recv_sems.at[3 + idx],
                device_id=(partners[idx],),
                device_id_type=pl.DeviceIdType.MESH,
            )
            rdma.start()
            rdma.wait()


    return pl.pallas_call(
        body,
        out_shape=jax.ShapeDtypeStruct((m, n), jnp.bfloat16),
        in_specs=[
            pl.BlockSpec(memory_space=pltpu.VMEM),
            pl.BlockSpec(memory_space=pltpu.VMEM),
        ],
        out_specs=pl.BlockSpec(memory_space=pltpu.VMEM),
        scratch_shapes=[
            pltpu.VMEM((rs_rows, n), jnp.bfloat16),
            pltpu.SemaphoreType.DMA((6,)),
            pltpu.SemaphoreType.DMA((6,)),
        ],
        compiler_params=pltpu.CompilerParams(collective_id=0),
    )(A, B)


# baseline (device time: 445431 ns/iter reference)
import jax
import jax.numpy as jnp
from jax import lax
from jax.experimental import pallas as pl
from jax.experimental.pallas import tpu as pltpu

N_DEV = 8


def kernel(A, B):
    A = A.astype(jnp.bfloat16)
    B = B.astype(jnp.bfloat16)
    m, _ = A.shape
    _, n = B.shape
    chunk = m // N_DEV
    rs_rows = m // 2 + m // 4 + m // 8

    def body(a_ref, b_ref, out_ref, recv_ref, send_sems, recv_sems):
        p = lax.axis_index("i")
        v = p ^ ((p >> 1) & 1)

        def to_logical(vq):
            return vq ^ ((vq >> 1) & 1)

        partners = [to_logical(v ^ (1 << b)) for b in range(3)]

        barrier = pltpu.get_barrier_semaphore()
        for b in range(3):
            pl.semaphore_signal(
                barrier, inc=1,
                device_id=(partners[b],),
                device_id_type=pl.DeviceIdType.MESH,
            )
        pl.semaphore_wait(barrier, 3)

        mm_chunk = m // 4
        for c in range(4):
            out_ref[c * mm_chunk:(c + 1) * mm_chunk, :] = jnp.dot(
                a_ref[c * mm_chunk:(c + 1) * mm_chunk, :],
                b_ref[...],
                preferred_element_type=jnp.float32,
            ).astype(jnp.bfloat16)

        cur = jnp.int32(0)
        recv_off = 0
        for idx, bit in enumerate((2, 1, 0)):
            half = m >> (idx + 1)
            mybit = (v >> bit) & 1
            keep = cur + mybit * half
            give = cur + (1 - mybit) * half
            rdma = pltpu.make_async_remote_copy(
                src_ref=out_ref.at[pl.ds(give, half), :],
                dst_ref=recv_ref.at[pl.ds(recv_off, half), :],
                send_sem=send_sems.at[idx],
                recv_sem=recv_sems.at[idx],
                device_id=(partners[bit],),
                device_id_type=pl.DeviceIdType.MESH,
            )
            rdma.start()
            rdma.wait()
            out_ref[pl.ds(keep, half), :] = (
                out_ref[pl.ds(keep, half), :]
                + recv_ref[pl.ds(recv_off, half), :]
            )
            cur = keep
            recv_off += half


        for idx in range(3):
            size = chunk << idx
            start = ((v >> idx) << idx) * chunk
            rdma = pltpu.make_async_remote_copy(
                src_ref=out_ref.at[pl.ds(start, size), :],
                dst_ref=out_ref.at[pl.ds(start, size), :],
                send_sem=send_sems.at[3 + idx],
                recv_sem=recv_sems.at[3 + idx],
                device_id=(partners[idx],),
                device_id_type=pl.DeviceIdType.MESH,
            )
            rdma.start()
            rdma.wait()


    return pl.pallas_call(
        body,
        out_shape=jax.ShapeDtypeStruct((m, n), jnp.bfloat16),
        in_specs=[
            pl.BlockSpec(memory_space=pltpu.VMEM),
            pl.BlockSpec(memory_space=pltpu.VMEM),
        ],
        out_specs=pl.BlockSpec(memory_space=pltpu.VMEM),
        scratch_shapes=[
            pltpu.VMEM((rs_rows, n), jnp.bfloat16),
            pltpu.SemaphoreType.DMA((6,)),
            pltpu.SemaphoreType.DMA((6,)),
        ],
        compiler_params=pltpu.CompilerParams(
            collective_id=0,
            vmem_limit_bytes=100 * 1024 * 1024,
        ),
    )(A, B)


# device time: 209025 ns/iter; 2.1310x vs baseline; 2.1310x over previous
import jax
import jax.numpy as jnp
from jax import lax
from jax.experimental import pallas as pl
from jax.experimental.pallas import tpu as pltpu

N_DEV = 8
S = 3

AXIS_ORDER = ((2, 1, 0), (1, 0, 2), (0, 2, 1))


def kernel(A, B):
    A = A.astype(jnp.bfloat16)
    B = B.astype(jnp.bfloat16)
    m, _ = A.shape
    _, n = B.shape
    w = n // S
    chunk = m // N_DEV
    rs_rows = m // 2 + m // 4 + m // 8
    RS_OFF = (0, m // 2, m // 2 + m // 4)

    def body(a_ref, b_ref, out_ref, recv_ref, send_sems, recv_sems):
        p = lax.axis_index("i")
        v = p ^ ((p >> 1) & 1)

        def to_logical(vq):
            return vq ^ ((vq >> 1) & 1)

        partners = [to_logical(v ^ (1 << b)) for b in range(3)]
        bits = [(v >> b) & 1 for b in range(3)]

        barrier = pltpu.get_barrier_semaphore()
        for b in range(3):
            pl.semaphore_signal(
                barrier, inc=1,
                device_id=(partners[b],),
                device_id_type=pl.DeviceIdType.MESH,
            )
        pl.semaphore_wait(barrier, 3)

        def cols(j):
            return pl.ds(j * w, w)

        def make_rs(j, s, cur):
            axis = AXIS_ORDER[j][s]
            half = m >> (s + 1)
            b = bits[axis]
            keep = cur + b * half
            give = cur + (1 - b) * half
            rd = pltpu.make_async_remote_copy(
                src_ref=out_ref.at[pl.ds(give, half), cols(j)],
                dst_ref=recv_ref.at[pl.ds(RS_OFF[s], half), cols(j)],
                send_sem=send_sems.at[j, s],
                recv_sem=recv_sems.at[j, s],
                device_id=(partners[axis],),
                device_id_type=pl.DeviceIdType.MESH,
            )
            rd.start()
            return rd, keep, half, RS_OFF[s]

        def make_ag(j, k, cur):
            size = chunk << k
            axis = AXIS_ORDER[j][2 - k]
            merged = cur - bits[axis] * size
            rd = pltpu.make_async_remote_copy(
                src_ref=out_ref.at[pl.ds(cur, size), cols(j)],
                dst_ref=out_ref.at[pl.ds(cur, size), cols(j)],
                send_sem=send_sems.at[j, 3 + k],
                recv_sem=recv_sems.at[j, 3 + k],
                device_id=(partners[axis],),
                device_id_type=pl.DeviceIdType.MESH,
            )
            rd.start()
            return rd, merged

        state = [None] * S
        for j in range(S):
            for c in range(2):
                rows = pl.ds(c * (m // 2), m // 2)
                out_ref[rows, cols(j)] = jnp.dot(
                    a_ref[rows, :],
                    b_ref[:, cols(j)],
                    preferred_element_type=jnp.float32,
                ).astype(jnp.bfloat16)
            state[j] = make_rs(j, 0, jnp.int32(0))

        for s in range(3):
            for j in range(S):
                rd, keep, half, off = state[j]
                rd.wait()
                out_ref[pl.ds(keep, half), cols(j)] = (
                    out_ref[pl.ds(keep, half), cols(j)]
                    + recv_ref[pl.ds(off, half), cols(j)]
                )
                if s < 2:
                    state[j] = make_rs(j, s + 1, keep)
                else:
                    state[j] = make_ag(j, 0, keep)

        for k in range(3):
            for j in range(S):
                rd, merged = state[j]
                rd.wait()
                if k < 2:
                    state[j] = make_ag(j, k + 1, merged)


    return pl.pallas_call(
        body,
        out_shape=jax.ShapeDtypeStruct((m, n), jnp.bfloat16),
        in_specs=[
            pl.BlockSpec(memory_space=pltpu.VMEM),
            pl.BlockSpec(memory_space=pltpu.VMEM),
        ],
        out_specs=pl.BlockSpec(memory_space=pltpu.VMEM),
        scratch_shapes=[
            pltpu.VMEM((rs_rows, n), jnp.bfloat16),
            pltpu.SemaphoreType.DMA((S, 6)),
            pltpu.SemaphoreType.DMA((S, 6)),
        ],
        compiler_params=pltpu.CompilerParams(
            collective_id=0,
            vmem_limit_bytes=100 * 1024 * 1024,
        ),
    )(A, B)
